# baseline (speedup 1.0000x reference)
import functools

import jax
import jax.numpy as jnp
from jax.experimental import pallas as pl
from jax.experimental.pallas import tpu as pltpu


def _make_body(nE, nT):
    def _body(x_ref, wfe_ref, bfe_ref, wexp_ref, bexp_ref, out_ref,
              feat_ref, log_ref):
        t = pl.program_id(0)

        @pl.when(jnp.logical_and(t % nE == 0, t < nT))
        def _trunk():
            acc = jnp.dot(x_ref[...], wfe_ref[...],
                          preferred_element_type=jnp.float32)
            feat_ref[...] = jnp.maximum(acc + bfe_ref[...], 0.0
                                        ).astype(jnp.bfloat16)

        @pl.when(t < nT)
        def _dot():
            log_ref[t % 2] = jnp.dot(feat_ref[...], wexp_ref[0],
                                     preferred_element_type=jnp.float32
                                     ) + bexp_ref[0]

        @pl.when(t > 0)
        def _epilogue():
            l = log_ref[(t + 1) % 2]
            out_ref[0] = jnp.maximum(l, 0.0) + jnp.log1p(jnp.exp(-jnp.abs(l)))

    return _body


@functools.partial(jax.jit, static_argnames=("bn",))
def _geer(x, W_fe, b_fe, W_exp, b_exp, bn=1024):
    n, d = x.shape
    ne, _, c = W_exp.shape
    bn = min(bn, n)
    nn = n // bn
    nt = nn * ne
    xb = x.astype(jnp.bfloat16)
    wfeb = W_fe.astype(jnp.bfloat16)
    wexpb = W_exp.astype(jnp.bfloat16)
    bfe2 = b_fe.reshape(1, d).astype(jnp.float32)
    bexp2 = b_exp.reshape(ne, 1, c).astype(jnp.float32)

    def dot_i(t):
        return jnp.minimum(t, nt - 1) // ne

    def dot_e(t):
        return jnp.minimum(t, nt - 1) % ne

    def epi_t(t):
        return jnp.maximum(t - 1, 0)

    return pl.pallas_call(
        _make_body(ne, nt),
        grid=(nt + 1,),
        in_specs=[
            pl.BlockSpec((bn, d), lambda t: (dot_i(t), 0)),
            pl.BlockSpec((d, d), lambda t: (0, 0)),
            pl.BlockSpec((1, d), lambda t: (0, 0)),
            pl.BlockSpec((1, d, c), lambda t: (dot_e(t), 0, 0)),
            pl.BlockSpec((1, 1, c), lambda t: (dot_e(t), 0, 0)),
        ],
        out_specs=pl.BlockSpec(
            (1, bn, c), lambda t: (epi_t(t) % ne, epi_t(t) // ne, 0)),
        out_shape=jax.ShapeDtypeStruct((ne, n, c), jnp.float32),
        scratch_shapes=[
            pltpu.VMEM((bn, d), jnp.bfloat16),
            pltpu.VMEM((2, bn, c), jnp.float32),
        ],
        compiler_params=pltpu.CompilerParams(
            dimension_semantics=("arbitrary",),
        ),
    )(xb, wfeb, bfe2, wexpb, bexp2)


def kernel(x, W_fe, b_fe, W_exp, b_exp):
    return _geer(x, W_fe, b_fe, W_exp, b_exp)
